# baseline (speedup 1.0000x reference)
import functools

import jax
import jax.numpy as jnp
from jax import lax
from jax.experimental import pallas as pl
from jax.experimental.pallas import tpu as pltpu
from jax.experimental.pallas import tpu_sc as plsc

N_NODES = 10000
D_HIST = 16
N_STEPS = 64
EQ_STEPS = 16

NSUB = 16
NCORES = 2
LANES = 16


def _tanh(x):
    e = jnp.exp(2.0 * x)
    return 1.0 - 2.0 / (e + 1.0)


def _make_kernel(n_pad, seg, win, e_cap, n_steps, eq_steps, interpret=False):
    n_chunks = seg // LANES
    w_chunks = win // LANES
    total_steps = n_steps + eq_steps

    mesh = plsc.VectorSubcoreMesh(
        core_axis_name="c", subcore_axis_name="s",
        num_cores=NCORES, num_subcores=NSUB)

    @functools.partial(
        pl.kernel,
        out_type=jax.ShapeDtypeStruct((n_steps, n_pad), jnp.float32),
        mesh=mesh,
        interpret=interpret,
        compiler_params=pltpu.CompilerParams(needs_layout_passes=False),
        scratch_types=[
            pltpu.VMEM((n_pad,), jnp.float32),
            pltpu.VMEM((e_cap,), jnp.int32),
            pltpu.VMEM((e_cap,), jnp.float32),
            pltpu.VMEM((win + LANES,), jnp.float32),
            pltpu.VMEM((seg,), jnp.int32),
            pltpu.VMEM((seg,), jnp.int32),
            pltpu.VMEM((seg,), jnp.float32),
            pltpu.VMEM((seg,), jnp.float32),
            pltpu.VMEM((LANES,), jnp.float32),
            pltpu.VMEM((LANES,), jnp.float32),
            pltpu.VMEM_SHARED((n_pad,), jnp.float32),
        ],
    )
    def k(a0_hbm, src_hbm, w_hbm, rp0_hbm, rp1_hbm, astart_hbm, nwin_hbm,
          out_hbm, a_local, src_c, w_c, cwin, rp0_v, rp1_v, acc, act,
          astart_v, nwin_v, act_sh):
        cid = lax.axis_index("c")
        sid = lax.axis_index("s")
        base = sid * seg

        pltpu.sync_copy(astart_hbm, astart_v)
        pltpu.sync_copy(nwin_hbm, nwin_v)
        lanes_i = lax.broadcasted_iota(jnp.int32, (LANES,), 0)
        msk = lanes_i == sid
        zero16f_ = jnp.zeros((LANES,), jnp.float32)
        a_start = pl.multiple_of(
            jnp.sum(jnp.where(msk, astart_v[...], zero16f_)).astype(jnp.int32),
            LANES)
        nw = jnp.sum(jnp.where(msk, nwin_v[...], zero16f_)).astype(jnp.int32)
        cached = nw * win <= e_cap

        pltpu.sync_copy(a0_hbm, a_local)
        pltpu.sync_copy(rp0_hbm.at[pl.ds(base, seg)], rp0_v)
        pltpu.sync_copy(rp1_hbm.at[pl.ds(base, seg)], rp1_v)

        @pl.when(cached)
        def _():
            pltpu.sync_copy(src_hbm.at[pl.ds(a_start, e_cap)], src_c)
            pltpu.sync_copy(w_hbm.at[pl.ds(a_start, e_cap)], w_c)

        zero16f = jnp.zeros((LANES,), jnp.float32)

        def step(t, carry_unused):
            def zero_body(i, _):
                acc[pl.ds(i * LANES, LANES)] = zero16f
                return 0
            lax.fori_loop(0, n_chunks, zero_body, 0)

            def win_body(w, carry):
                @pl.when(jnp.logical_not(cached))
                def _():
                    off = pl.multiple_of(a_start + w * win, LANES)
                    pltpu.sync_copy(src_hbm.at[pl.ds(off, win)],
                                    src_c.at[pl.ds(0, win)])
                    pltpu.sync_copy(w_hbm.at[pl.ds(off, win)],
                                    w_c.at[pl.ds(0, win)])
                woff = jnp.where(cached, w * win, 0)

                unroll = 8

                def chunk_body(ch4, cy):
                    incs = []
                    for u in range(unroll):
                        ch = ch4 * unroll + u
                        o = woff + ch * LANES
                        idx = src_c[pl.ds(o, LANES)]
                        wv = w_c[pl.ds(o, LANES)]
                        vals = plsc.load_gather(a_local, [idx])
                        msg = wv * vals
                        incs.append((ch, msg, plsc.cumsum(msg)))
                    for ch, msg, inc in incs:
                        cwin[pl.ds(ch * LANES, LANES)] = inc - msg + cy
                        tot = lax.squeeze(lax.slice(inc, (LANES - 1,), (LANES,)),
                                          (0,))
                        cy = cy + tot
                    return cy
                carry = lax.fori_loop(0, w_chunks // unroll, chunk_body, carry)
                cwin[pl.ds(win, LANES)] = carry

                wlo = w * win

                def node_body(nc4, _):
                    for u in range(4):
                        o = (nc4 * 4 + u) * LANES
                        p0 = rp0_v[pl.ds(o, LANES)]
                        p1 = rp1_v[pl.ds(o, LANES)]
                        q0 = jnp.clip(p0, wlo, wlo + win) - wlo
                        q1 = jnp.clip(p1, wlo, wlo + win) - wlo
                        g1 = plsc.load_gather(cwin, [q1])
                        g0 = plsc.load_gather(cwin, [q0])
                        acc[pl.ds(o, LANES)] = acc[pl.ds(o, LANES)] + (g1 - g0)
                    return 0
                lax.fori_loop(0, n_chunks // 4, node_body, 0)
                return carry

            lax.fori_loop(0, nw, win_body, zero16f)

            def act_body(nc, _):
                o = nc * LANES
                agg = acc[pl.ds(o, LANES)]
                prev = a_local[pl.ds(base + o, LANES)]
                act[pl.ds(o, LANES)] = _tanh(agg + 0.9 * prev)
                return 0
            lax.fori_loop(0, n_chunks, act_body, 0)

            @pl.when(jnp.logical_and(t >= eq_steps, cid == 0))
            def _():
                pltpu.sync_copy(act, out_hbm.at[t - eq_steps, pl.ds(base, seg)])

            pltpu.sync_copy(act, act_sh.at[pl.ds(base, seg)])
            plsc.subcore_barrier()
            pltpu.sync_copy(act_sh, a_local)
            plsc.subcore_barrier()
            return 0

        lax.fori_loop(0, total_steps, step, 0)

    return k


def _prepare(x, edge_index, W, n_pad, seg, win, e_cap):
    n = x.shape[0]
    src = edge_index[0]
    dst = edge_index[1]
    try:
        from jax.experimental.compute_on import compute_on

        @compute_on("tpu_sparsecore")
        @jax.jit
        def _sc_argsort(k):
            return jnp.argsort(k)

        order = _sc_argsort(dst)
    except Exception:
        order = jnp.argsort(dst)
    src_s = jnp.take(src, order).astype(jnp.int32)
    dst_s = jnp.take(dst, order).astype(jnp.int32)
    w_s = jnp.take(W, order)

    ptr = jnp.searchsorted(dst_s, jnp.arange(n_pad + 1, dtype=jnp.int32),
                           side="left").astype(jnp.int32)
    seg_bound = ptr[::seg]
    a_start = (seg_bound[:-1] // LANES) * LANES
    seg_end = seg_bound[1:]
    seg_len = seg_end - a_start
    nwin = (seg_len + win - 1) // win
    a_start_f = a_start.astype(jnp.float32)
    nwin_f = nwin.astype(jnp.float32)

    a_start_per_node = jnp.repeat(a_start, seg)
    rp0 = ptr[:-1] - a_start_per_node
    rp1 = ptr[1:] - a_start_per_node

    pad_e = e_cap + LANES
    src_g = jnp.concatenate([src_s, jnp.zeros((pad_e,), jnp.int32)])
    w_g = jnp.concatenate([w_s, jnp.zeros((pad_e,), jnp.float32)])
    a0 = jnp.concatenate(
        [x[:, -1], jnp.zeros((n_pad - n,), jnp.float32)])
    return a0, src_g, w_g, rp0, rp1, a_start_f, nwin_f


def kernel(x, edge_index, W):
    n_pad = 10240
    seg = n_pad // NSUB
    win = 4096
    e_cap = 32768
    a0, src_g, w_g, rp0, rp1, a_start, nwin = _prepare(
        x, edge_index, W, n_pad, seg, win, e_cap)
    k = _make_kernel(n_pad, seg, win, e_cap, N_STEPS, EQ_STEPS)
    out = k(a0, src_g, w_g, rp0, rp1, a_start, nwin)
    return out[:, :N_NODES]


# speedup vs baseline: 1.0411x; 1.0411x over previous
import functools

import jax
import jax.numpy as jnp
from jax import lax
from jax.experimental import pallas as pl
from jax.experimental.pallas import tpu as pltpu
from jax.experimental.pallas import tpu_sc as plsc

N_NODES = 10000
D_HIST = 16
N_STEPS = 64
EQ_STEPS = 16

NSUB = 16
NCORES = 2
LANES = 16


def _tanh(x):
    e = jnp.exp(2.0 * x)
    return 1.0 - 2.0 / (e + 1.0)


def _make_kernel(n_pad, seg, win, e_cap, n_steps, eq_steps, interpret=False):
    n_chunks = seg // LANES
    w_chunks = win // LANES
    total_steps = n_steps + eq_steps

    mesh = plsc.VectorSubcoreMesh(
        core_axis_name="c", subcore_axis_name="s",
        num_cores=NCORES, num_subcores=NSUB)

    @functools.partial(
        pl.kernel,
        out_type=jax.ShapeDtypeStruct((n_steps, n_pad), jnp.float32),
        mesh=mesh,
        interpret=interpret,
        compiler_params=pltpu.CompilerParams(needs_layout_passes=False),
        scratch_types=[
            pltpu.VMEM((n_pad,), jnp.float32),
            pltpu.VMEM((e_cap,), jnp.int32),
            pltpu.VMEM((e_cap,), jnp.float32),
            pltpu.VMEM((win + LANES,), jnp.float32),
            pltpu.VMEM((seg,), jnp.int32),
            pltpu.VMEM((seg,), jnp.int32),
            pltpu.VMEM((seg,), jnp.float32),
            pltpu.VMEM((seg,), jnp.float32),
            pltpu.VMEM((LANES,), jnp.float32),
            pltpu.VMEM((LANES,), jnp.float32),
            pltpu.VMEM_SHARED((n_pad,), jnp.float32),
        ],
    )
    def k(a0_hbm, src_hbm, w_hbm, rp0_hbm, rp1_hbm, astart_hbm, nwin_hbm,
          out_hbm, a_local, src_c, w_c, cwin, rp0_v, rp1_v, acc, act,
          astart_v, nwin_v, act_sh):
        cid = lax.axis_index("c")
        sid = lax.axis_index("s")
        base = sid * seg

        pltpu.sync_copy(astart_hbm, astart_v)
        pltpu.sync_copy(nwin_hbm, nwin_v)
        lanes_i = lax.broadcasted_iota(jnp.int32, (LANES,), 0)
        msk = lanes_i == sid
        zero16f_ = jnp.zeros((LANES,), jnp.float32)
        a_start = pl.multiple_of(
            jnp.sum(jnp.where(msk, astart_v[...], zero16f_)).astype(jnp.int32),
            LANES)
        nw = jnp.sum(jnp.where(msk, nwin_v[...], zero16f_)).astype(jnp.int32)
        cached = nw * win <= e_cap

        pltpu.sync_copy(a0_hbm, a_local)
        pltpu.sync_copy(rp0_hbm.at[pl.ds(base, seg)], rp0_v)
        pltpu.sync_copy(rp1_hbm.at[pl.ds(base, seg)], rp1_v)

        @pl.when(cached)
        def _():
            pltpu.sync_copy(src_hbm.at[pl.ds(a_start, e_cap)], src_c)
            pltpu.sync_copy(w_hbm.at[pl.ds(a_start, e_cap)], w_c)

        zero16f = jnp.zeros((LANES,), jnp.float32)

        def step(t, carry_unused):
            def zero_body(i, _):
                acc[pl.ds(i * LANES, LANES)] = zero16f
                return 0
            lax.fori_loop(0, n_chunks, zero_body, 0)

            def win_body(w, carry):
                @pl.when(jnp.logical_not(cached))
                def _():
                    off = pl.multiple_of(a_start + w * win, LANES)
                    pltpu.sync_copy(src_hbm.at[pl.ds(off, win)],
                                    src_c.at[pl.ds(0, win)])
                    pltpu.sync_copy(w_hbm.at[pl.ds(off, win)],
                                    w_c.at[pl.ds(0, win)])
                woff = jnp.where(cached, w * win, 0)

                unroll = 8

                def chunk_body(ch4, cy):
                    incs = []
                    for u in range(unroll):
                        ch = ch4 * unroll + u
                        o = woff + ch * LANES
                        idx = src_c[pl.ds(o, LANES)]
                        wv = w_c[pl.ds(o, LANES)]
                        vals = plsc.load_gather(a_local, [idx])
                        msg = wv * vals
                        incs.append((ch, msg, plsc.cumsum(msg)))
                    for ch, msg, inc in incs:
                        cwin[pl.ds(ch * LANES, LANES)] = inc - msg + cy
                        tot = lax.squeeze(lax.slice(inc, (LANES - 1,), (LANES,)),
                                          (0,))
                        cy = cy + tot
                    return cy
                carry = lax.fori_loop(0, w_chunks // unroll, chunk_body, carry)
                cwin[pl.ds(win, LANES)] = carry

                wlo = w * win

                def node_body(nc4, _):
                    for u in range(4):
                        o = (nc4 * 4 + u) * LANES
                        p0 = rp0_v[pl.ds(o, LANES)]
                        p1 = rp1_v[pl.ds(o, LANES)]
                        q0 = jnp.clip(p0, wlo, wlo + win) - wlo
                        q1 = jnp.clip(p1, wlo, wlo + win) - wlo
                        g1 = plsc.load_gather(cwin, [q1])
                        g0 = plsc.load_gather(cwin, [q0])
                        acc[pl.ds(o, LANES)] = acc[pl.ds(o, LANES)] + (g1 - g0)
                    return 0
                lax.fori_loop(0, n_chunks // 4, node_body, 0)
                return carry

            lax.fori_loop(0, nw, win_body, zero16f)

            def act_body(nc, _):
                o = nc * LANES
                agg = acc[pl.ds(o, LANES)]
                prev = a_local[pl.ds(base + o, LANES)]
                act[pl.ds(o, LANES)] = _tanh(agg + 0.9 * prev)
                return 0
            lax.fori_loop(0, n_chunks, act_body, 0)

            @pl.when(jnp.logical_and(t >= eq_steps, cid == 0))
            def _():
                pltpu.sync_copy(act, out_hbm.at[t - eq_steps, pl.ds(base, seg)])

            pltpu.sync_copy(act, act_sh.at[pl.ds(base, seg)])
            plsc.subcore_barrier()
            pltpu.sync_copy(act_sh, a_local)
            plsc.subcore_barrier()
            return 0

        lax.fori_loop(0, total_steps, step, 0)

    return k


def _prepare(x, edge_index, W, n_pad, seg, win, e_cap):
    n = x.shape[0]
    src = edge_index[0]
    dst = edge_index[1]
    dst_s, src_s, w_s = lax.sort(
        (dst.astype(jnp.int32), src.astype(jnp.int32), W),
        num_keys=1, is_stable=False)

    ptr = jnp.searchsorted(dst_s, jnp.arange(n_pad + 1, dtype=jnp.int32),
                           side="left").astype(jnp.int32)
    seg_bound = ptr[::seg]
    a_start = (seg_bound[:-1] // LANES) * LANES
    seg_end = seg_bound[1:]
    seg_len = seg_end - a_start
    nwin = (seg_len + win - 1) // win
    a_start_f = a_start.astype(jnp.float32)
    nwin_f = nwin.astype(jnp.float32)

    a_start_per_node = jnp.repeat(a_start, seg)
    rp0 = ptr[:-1] - a_start_per_node
    rp1 = ptr[1:] - a_start_per_node

    pad_e = e_cap + LANES
    src_g = jnp.concatenate([src_s, jnp.zeros((pad_e,), jnp.int32)])
    w_g = jnp.concatenate([w_s, jnp.zeros((pad_e,), jnp.float32)])
    a0 = jnp.concatenate(
        [x[:, -1], jnp.zeros((n_pad - n,), jnp.float32)])
    return a0, src_g, w_g, rp0, rp1, a_start_f, nwin_f


def kernel(x, edge_index, W):
    n_pad = 10240
    seg = n_pad // NSUB
    win = 4096
    e_cap = 32768
    a0, src_g, w_g, rp0, rp1, a_start, nwin = _prepare(
        x, edge_index, W, n_pad, seg, win, e_cap)
    k = _make_kernel(n_pad, seg, win, e_cap, N_STEPS, EQ_STEPS)
    out = k(a0, src_g, w_g, rp0, rp1, a_start, nwin)
    return out[:, :N_NODES]


# speedup vs baseline: 1.6122x; 1.5486x over previous
import functools

import jax
import jax.numpy as jnp
from jax import lax
from jax.experimental import pallas as pl
from jax.experimental.pallas import tpu as pltpu
from jax.experimental.pallas import tpu_sc as plsc

N_NODES = 10000
N_EDGES = 320000
N_STEPS = 64
EQ_STEPS = 16

NSUB = 16
NCORES = 2
LANES = 16


def _tanh(x):
    e = jnp.exp(2.0 * x)
    return 1.0 - 2.0 / (e + 1.0)


def _make_kernel(n_pad, e_per_w, n_steps, eq_steps):
    seg = n_pad // NSUB
    n_chunks = seg // LANES
    e_chunks = e_per_w // LANES
    total_steps = n_steps + eq_steps

    mesh = plsc.VectorSubcoreMesh(
        core_axis_name="c", subcore_axis_name="s",
        num_cores=NCORES, num_subcores=NSUB)

    @functools.partial(
        pl.kernel,
        out_type=jax.ShapeDtypeStruct((n_steps, n_pad), jnp.float32),
        mesh=mesh,
        compiler_params=pltpu.CompilerParams(needs_layout_passes=False),
        scratch_types=[
            pltpu.VMEM((n_pad,), jnp.float32),
            pltpu.VMEM((e_per_w,), jnp.int32),
            pltpu.VMEM((e_per_w,), jnp.int32),
            pltpu.VMEM((e_per_w,), jnp.float32),
            pltpu.VMEM((n_pad,), jnp.float32),
            pltpu.VMEM((NSUB, seg), jnp.float32),
            pltpu.VMEM((seg,), jnp.float32),
            pltpu.VMEM_SHARED((NSUB, n_pad), jnp.float32),
            pltpu.VMEM_SHARED((n_pad,), jnp.float32),
        ],
    )
    def k(a0_hbm, src_hbm, dst_hbm, w_hbm, out_hbm,
          a_local, src_v, dst_v, w_v, acc, red, act, accs_sh, act_sh):
        cid = lax.axis_index("c")
        sid = lax.axis_index("s")
        base = sid * seg
        ebase = sid * e_per_w

        pltpu.sync_copy(a0_hbm, a_local)
        pltpu.sync_copy(src_hbm.at[pl.ds(ebase, e_per_w)], src_v)
        pltpu.sync_copy(dst_hbm.at[pl.ds(ebase, e_per_w)], dst_v)
        pltpu.sync_copy(w_hbm.at[pl.ds(ebase, e_per_w)], w_v)

        zero16f = jnp.zeros((LANES,), jnp.float32)

        def step(t, _unused):
            def zero_body(i, _):
                acc[pl.ds(i * LANES * 8, LANES)] = zero16f
                acc[pl.ds(i * LANES * 8 + 16, LANES)] = zero16f
                acc[pl.ds(i * LANES * 8 + 32, LANES)] = zero16f
                acc[pl.ds(i * LANES * 8 + 48, LANES)] = zero16f
                acc[pl.ds(i * LANES * 8 + 64, LANES)] = zero16f
                acc[pl.ds(i * LANES * 8 + 80, LANES)] = zero16f
                acc[pl.ds(i * LANES * 8 + 96, LANES)] = zero16f
                acc[pl.ds(i * LANES * 8 + 112, LANES)] = zero16f
                return 0
            lax.fori_loop(0, n_pad // (LANES * 8), zero_body, 0)

            unroll = 8

            def edge_body(cU, _):
                for u in range(unroll):
                    o = (cU * unroll + u) * LANES
                    idx = src_v[pl.ds(o, LANES)]
                    d = dst_v[pl.ds(o, LANES)]
                    wv = w_v[pl.ds(o, LANES)]
                    vals = plsc.load_gather(a_local, [idx])
                    plsc.addupdate_scatter(acc, [d], wv * vals)
                return 0
            lax.fori_loop(0, e_chunks // unroll, edge_body, 0)

            pltpu.sync_copy(acc, accs_sh.at[sid])
            plsc.subcore_barrier()
            pltpu.sync_copy(accs_sh.at[:, pl.ds(base, seg)], red)

            def act_body(nc, _):
                o = nc * LANES
                agg = red[0, pl.ds(o, LANES)]
                for r in range(1, NSUB):
                    agg = agg + red[r, pl.ds(o, LANES)]
                prev = a_local[pl.ds(base + o, LANES)]
                act[pl.ds(o, LANES)] = _tanh(agg + 0.9 * prev)
                return 0
            lax.fori_loop(0, n_chunks, act_body, 0)

            @pl.when(jnp.logical_and(t >= eq_steps, cid == 0))
            def _():
                pltpu.sync_copy(act, out_hbm.at[t - eq_steps, pl.ds(base, seg)])

            pltpu.sync_copy(act, act_sh.at[pl.ds(base, seg)])
            plsc.subcore_barrier()
            pltpu.sync_copy(act_sh, a_local)
            plsc.subcore_barrier()
            return 0

        lax.fori_loop(0, total_steps, step, 0)

    return k


def kernel(x, edge_index, W):
    n_pad = 10240
    e_per_w = N_EDGES // NSUB
    a0 = jnp.concatenate(
        [x[:, -1], jnp.zeros((n_pad - N_NODES,), jnp.float32)])
    src = edge_index[0].astype(jnp.int32)
    dst = edge_index[1].astype(jnp.int32)
    k = _make_kernel(n_pad, e_per_w, N_STEPS, EQ_STEPS)
    out = k(a0, src, dst, W)
    return out[:, :N_NODES]
